# baseline (speedup 1.0000x reference)
import math

import jax
import jax.numpy as jnp
from jax import lax
from jax.experimental import pallas as pl
from jax.experimental.pallas import tpu as pltpu
from jax.experimental.pallas import tpu_sc as plsc

_EPS = 1e-12
_F32 = jnp.float32
_BF16 = jnp.bfloat16


def _dot_t(a, b):
    return lax.dot_general(a, b, (((1,), (1,)), ((), ())),
                           preferred_element_type=_F32)


def _gelu(v):
    inner = math.sqrt(2.0 / math.pi) * (v + 0.044715 * v * v * v)
    return 0.5 * v * (1.0 + jnp.tanh(inner))


def kernel(x, W1w, W1b, W2w, W2b, Wfc, bfc, Wproj, bproj, Wr, br, g1, be1, g2, be2):
    Bb, N, D = x.shape
    E, F, _ = Wfc.shape
    H = 12
    DH = D // H
    K = 2
    TILE = 512
    NT = N // TILE
    ND = float(N * D)
    scale = 1.0 / math.sqrt(float(N))

    x2 = x.reshape(N, D)
    b1r = W1b.reshape(1, 3 * D)
    b2r = W2b.reshape(1, D)
    g1r, be1r = g1.reshape(1, D), be1.reshape(1, D)
    g2r, be2r = g2.reshape(1, D), be2.reshape(1, D)
    brr = br.reshape(1, E)

    def _front(x_ref, w1_ref, b1_ref, w2_ref, b2_ref,
               g1_ref, be1_ref, wr_ref, br_ref,
               y_ref, cs_ref, css_ref, stats_ref, lg_ref, v_s, s_s):
        g = pl.program_id(0)

        @pl.when(g == 0)
        def _():
            s_s[...] = jnp.zeros_like(s_s)

        @pl.when(g < NT)
        def _():
            xp = _dot_t(x_ref[...], w1_ref[...]) + b1_ref[...]
            base = pl.multiple_of(g * TILE, TILE)
            for h in range(H):
                v_s[h, pl.ds(base, TILE), :] = \
                    xp[:, 2 * D + h * DH:2 * D + (h + 1) * DH]
                qh = xp[:, h * DH:(h + 1) * DH]
                kh = xp[:, D + h * DH:D + (h + 1) * DH]
                s_s[h] += lax.dot_general(qh, kh, (((0,), (0,)), ((), ())),
                                          preferred_element_type=_F32)

        @pl.when(g >= NT)
        def _():
            i = g - NT
            parts = []
            for t in range(3):
                hh = 3 * i + t
                w = jax.nn.softmax(s_s[hh] * scale, axis=-1)
                parts.append(
                    lax.dot_general(w, v_s[hh], (((1,), (1,)), ((), ())),
                                    preferred_element_type=_F32))
            r = jnp.concatenate(parts, axis=0)
            att = jnp.reshape(r, (TILE, D))
            y = _dot_t(att, w2_ref[...]) + b2_ref[...] + x_ref[...]
            y_ref[...] = y

            @pl.when(i == 0)
            def _():
                cs_ref[...] = jnp.zeros_like(cs_ref)
                css_ref[...] = jnp.zeros_like(css_ref)

            cs_ref[...] += jnp.sum(y, axis=0, keepdims=True)
            css_ref[...] += jnp.sum(y * y, axis=0, keepdims=True)

            @pl.when(i == NT - 1)
            def _():
                total = jnp.sum(cs_ref[...])
                mu = total / ND
                ssq = jnp.sum(css_ref[...])
                var = (ssq - ND * mu * mu) / (ND - 1.0)
                rstd = 1.0 / jnp.sqrt(var + _EPS)
                colmean_x1 = ((cs_ref[...] / N) - mu) * rstd * g1_ref[...] \
                    + be1_ref[...]
                logits = _dot_t(colmean_x1, wr_ref[...]) + br_ref[...]
                probs = jax.nn.softmax(logits, axis=-1)
                pad = jnp.full((1, 16 - E), -1.0, _F32)
                lg_ref[...] = jnp.concatenate([probs, pad], axis=1)
                iota2 = lax.broadcasted_iota(jnp.int32, (1, 2), 1)
                stats_ref[...] = jnp.where(iota2 == 0, mu, rstd)

    def _xtile(g):
        return (jnp.where(g < NT, g, g - NT), 0)

    y, cs, css, stats2, lg16 = pl.pallas_call(
        _front,
        grid=(2 * NT,),
        in_specs=[
            pl.BlockSpec((TILE, D), _xtile),
            pl.BlockSpec((3 * D, D), lambda g: (0, 0)),
            pl.BlockSpec((1, 3 * D), lambda g: (0, 0)),
            pl.BlockSpec((D, D), lambda g: (0, 0)),
            pl.BlockSpec((1, D), lambda g: (0, 0)),
            pl.BlockSpec((1, D), lambda g: (0, 0)),
            pl.BlockSpec((1, D), lambda g: (0, 0)),
            pl.BlockSpec((E, D), lambda g: (0, 0)),
            pl.BlockSpec((1, E), lambda g: (0, 0)),
        ],
        out_specs=[
            pl.BlockSpec((TILE, D), _xtile),
            pl.BlockSpec((1, D), lambda g: (0, 0)),
            pl.BlockSpec((1, D), lambda g: (0, 0)),
            pl.BlockSpec((1, 2), lambda g: (0, 0)),
            pl.BlockSpec((1, 16), lambda g: (0, 0)),
        ],
        out_shape=[
            jax.ShapeDtypeStruct((N, D), _F32),
            jax.ShapeDtypeStruct((1, D), _F32),
            jax.ShapeDtypeStruct((1, D), _F32),
            jax.ShapeDtypeStruct((1, 2), _F32),
            jax.ShapeDtypeStruct((1, 16), _F32),
        ],
        scratch_shapes=[
            pltpu.VMEM((H, N, DH), _F32),
            pltpu.VMEM((H, DH, DH), _F32),
        ],
    )(x2, W1w, b1r, W2w, b2r, g1r, be1r, Wr, brr)

    def _router_sc(probs_hbm, vals_hbm, idx_hbm, pr_v, vals_v, idx_v):
        @pl.when((lax.axis_index("c") == 0) & (lax.axis_index("s") == 0))
        def _():
            pltpu.sync_copy(probs_hbm, pr_v)
            p = pr_v[...]
            ii = lax.iota(jnp.int32, 16)
            sp, si = plsc.sort_key_val(p, ii, descending=True)
            vals_v[...] = sp
            idx_v[...] = si
            pltpu.sync_copy(vals_v, vals_hbm)
            pltpu.sync_copy(idx_v.at[pl.ds(0, 8)], idx_hbm)

    vals16, idx8 = pl.kernel(
        _router_sc,
        out_type=[jax.ShapeDtypeStruct((16,), _F32),
                  jax.ShapeDtypeStruct((8,), jnp.int32)],
        mesh=plsc.VectorSubcoreMesh(core_axis_name="c", subcore_axis_name="s"),
        scratch_types=[pltpu.VMEM((16,), _F32),
                       pltpu.VMEM((16,), _F32),
                       pltpu.VMEM((16,), jnp.int32)],
        compiler_params=pltpu.CompilerParams(needs_layout_passes=False),
    )(lg16.reshape(16))

    vals2 = vals16.reshape(1, 16)
    bfc3 = bfc.reshape(E, 1, F)
    bproj3 = bproj.reshape(E, 1, D)

    FT = 1536
    NF = F // FT
    GE = K * NF

    def _tail(idx_ref, y_ref, cs_ref, css_ref, stats_ref, vals_ref,
              g1_ref, be1_ref, g2_ref, be2_ref,
              wfc_ref, bfc_ref, wproj_ref, bproj_ref, o_ref, m_s):
        g = pl.program_id(0)

        @pl.when(g == 0)
        def _():
            m_s[...] = jnp.zeros_like(m_s)

        @pl.when(g < GE)
        def _():
            j = g // NF
            f = g % NF
            mu = stats_ref[:, 0:1]
            rstd = stats_ref[:, 1:2]
            val = jnp.where(j == 0, vals_ref[:, 0:1], vals_ref[:, 1:2])
            yj = jnp.where(j == 0, y_ref[0:1, :], y_ref[1:2, :])
            x1j = (yj - mu) * rstd * g1_ref[...] + be1_ref[...]
            h = _gelu(_dot_t(x1j, wfc_ref[0]) + bfc_ref[0])
            o = _dot_t(h, wproj_ref[0])
            m_s[...] += val * o

            @pl.when(f == 0)
            def _():
                m_s[...] += val * bproj_ref[0]

        @pl.when(g >= GE)
        def _():
            total = jnp.sum(cs_ref[...])
            mu1 = total / ND
            ssq = jnp.sum(css_ref[...])
            var1 = (ssq - ND * mu1 * mu1) / (ND - 1.0)
            rstd1 = 1.0 / jnp.sqrt(var1 + _EPS)
            a = rstd1 * g1_ref[...]
            c = be1_ref[...] + m_s[...] - mu1 * a
            sz = jnp.sum(a * cs_ref[...] + N * c)
            szz = jnp.sum(a * a * css_ref[...] + 2.0 * a * c * cs_ref[...]
                          + N * c * c)
            mu2 = sz / ND
            var2 = (szz - ND * mu2 * mu2) / (ND - 1.0)
            rstd2 = 1.0 / jnp.sqrt(var2 + _EPS)
            z = y_ref[...] * a + c
            o_ref[...] = (z - mu2) * rstd2 * g2_ref[...] + be2_ref[...]

    def _ytile(g, idx):
        return (jnp.where(g < GE, 0, g - GE), 0)

    def _jj(g):
        return jnp.where(g < GE, g // NF, K - 1)

    def _ff(g):
        return jnp.where(g < GE, g % NF, NF - 1)

    out = pl.pallas_call(
        _tail,
        grid_spec=pltpu.PrefetchScalarGridSpec(
            num_scalar_prefetch=1,
            grid=(GE + NT,),
            in_specs=[
                pl.BlockSpec((TILE, D), _ytile),
                pl.BlockSpec((1, D), lambda g, idx: (0, 0)),
                pl.BlockSpec((1, D), lambda g, idx: (0, 0)),
                pl.BlockSpec((1, 2), lambda g, idx: (0, 0)),
                pl.BlockSpec((1, 16), lambda g, idx: (0, 0)),
                pl.BlockSpec((1, D), lambda g, idx: (0, 0)),
                pl.BlockSpec((1, D), lambda g, idx: (0, 0)),
                pl.BlockSpec((1, D), lambda g, idx: (0, 0)),
                pl.BlockSpec((1, D), lambda g, idx: (0, 0)),
                pl.BlockSpec((1, FT, D),
                             lambda g, idx: (idx[_jj(g)], _ff(g), 0)),
                pl.BlockSpec((1, 1, FT),
                             lambda g, idx: (idx[_jj(g)], 0, _ff(g))),
                pl.BlockSpec((1, D, FT),
                             lambda g, idx: (idx[_jj(g)], 0, _ff(g))),
                pl.BlockSpec((1, 1, D),
                             lambda g, idx: (idx[_jj(g)], 0, 0)),
            ],
            out_specs=pl.BlockSpec((TILE, D), _ytile),
            scratch_shapes=[pltpu.VMEM((1, D), _F32)],
        ),
        out_shape=jax.ShapeDtypeStruct((N, D), _F32),
    )(idx8, y, cs, css, stats2, vals2, g1r, be1r, g2r, be2r,
      Wfc, bfc3, Wproj, bproj3)

    return out.reshape(Bb, N, D)


# speedup vs baseline: 1.0174x; 1.0174x over previous
import math

import jax
import jax.numpy as jnp
from jax import lax
from jax.experimental import pallas as pl
from jax.experimental.pallas import tpu as pltpu
from jax.experimental.pallas import tpu_sc as plsc

_EPS = 1e-12
_F32 = jnp.float32
_BF16 = jnp.bfloat16


def _dot_t(a, b):
    return lax.dot_general(a, b, (((1,), (1,)), ((), ())),
                           preferred_element_type=_F32)


def _gelu(v):
    inner = math.sqrt(2.0 / math.pi) * (v + 0.044715 * v * v * v)
    return 0.5 * v * (1.0 + jnp.tanh(inner))


def kernel(x, W1w, W1b, W2w, W2b, Wfc, bfc, Wproj, bproj, Wr, br, g1, be1, g2, be2):
    Bb, N, D = x.shape
    E, F, _ = Wfc.shape
    H = 12
    DH = D // H
    K = 2
    TILE = 512
    NT = N // TILE
    ND = float(N * D)
    scale = 1.0 / math.sqrt(float(N))

    x2 = x.reshape(N, D)
    b1r = W1b.reshape(1, 3 * D)
    b2r = W2b.reshape(1, D)
    g1r, be1r = g1.reshape(1, D), be1.reshape(1, D)
    g2r, be2r = g2.reshape(1, D), be2.reshape(1, D)
    brr = br.reshape(1, E)

    def _front(x_ref, w1_ref, b1_ref, w2_ref, b2_ref,
               g1_ref, be1_ref, wr_ref, br_ref,
               y_ref, cs_ref, css_ref, stats_ref, lg_ref, v_s, s_s):
        g = pl.program_id(0)

        @pl.when(g == 0)
        def _():
            s_s[...] = jnp.zeros_like(s_s)

        @pl.when(g < NT)
        def _():
            xp = _dot_t(x_ref[...], w1_ref[...]) + b1_ref[...]
            base = pl.multiple_of(g * TILE, TILE)
            for h in range(H):
                v_s[h, pl.ds(base, TILE), :] = \
                    xp[:, 2 * D + h * DH:2 * D + (h + 1) * DH]
                qh = xp[:, h * DH:(h + 1) * DH]
                kh = xp[:, D + h * DH:D + (h + 1) * DH]
                s_s[h] += lax.dot_general(qh, kh, (((0,), (0,)), ((), ())),
                                          preferred_element_type=_F32)

        @pl.when(g >= NT)
        def _():
            i = g - NT
            parts = []
            for t in range(3):
                hh = 3 * i + t
                w = jax.nn.softmax(s_s[hh] * scale, axis=-1)
                parts.append(
                    lax.dot_general(w, v_s[hh], (((1,), (1,)), ((), ())),
                                    preferred_element_type=_F32))
            r = jnp.concatenate(parts, axis=0)
            att = jnp.reshape(r, (TILE, D))
            y = _dot_t(att, w2_ref[...]) + b2_ref[...] + x_ref[...]
            y_ref[...] = y

            @pl.when(i == 0)
            def _():
                cs_ref[...] = jnp.zeros_like(cs_ref)
                css_ref[...] = jnp.zeros_like(css_ref)

            cs_ref[...] += jnp.sum(y, axis=0, keepdims=True)
            css_ref[...] += jnp.sum(y * y, axis=0, keepdims=True)

            @pl.when(i == NT - 1)
            def _():
                total = jnp.sum(cs_ref[...])
                mu = total / ND
                ssq = jnp.sum(css_ref[...])
                var = (ssq - ND * mu * mu) / (ND - 1.0)
                rstd = 1.0 / jnp.sqrt(var + _EPS)
                colmean_x1 = ((cs_ref[...] / N) - mu) * rstd * g1_ref[...] \
                    + be1_ref[...]
                logits = _dot_t(colmean_x1, wr_ref[...]) + br_ref[...]
                probs = jax.nn.softmax(logits, axis=-1)
                pad = jnp.full((1, 16 - E), -1.0, _F32)
                lg_ref[...] = jnp.concatenate([probs, pad], axis=1)
                iota2 = lax.broadcasted_iota(jnp.int32, (1, 2), 1)
                stats_ref[...] = jnp.where(iota2 == 0, mu, rstd)

    def _xtile(g):
        return (jnp.where(g < NT, g, g - NT), 0)

    y, cs, css, stats2, lg16 = pl.pallas_call(
        _front,
        grid=(2 * NT,),
        in_specs=[
            pl.BlockSpec((TILE, D), _xtile),
            pl.BlockSpec((3 * D, D), lambda g: (0, 0)),
            pl.BlockSpec((1, 3 * D), lambda g: (0, 0)),
            pl.BlockSpec((D, D), lambda g: (0, 0)),
            pl.BlockSpec((1, D), lambda g: (0, 0)),
            pl.BlockSpec((1, D), lambda g: (0, 0)),
            pl.BlockSpec((1, D), lambda g: (0, 0)),
            pl.BlockSpec((E, D), lambda g: (0, 0)),
            pl.BlockSpec((1, E), lambda g: (0, 0)),
        ],
        out_specs=[
            pl.BlockSpec((TILE, D), _xtile),
            pl.BlockSpec((1, D), lambda g: (0, 0)),
            pl.BlockSpec((1, D), lambda g: (0, 0)),
            pl.BlockSpec((1, 2), lambda g: (0, 0)),
            pl.BlockSpec((1, 16), lambda g: (0, 0)),
        ],
        out_shape=[
            jax.ShapeDtypeStruct((N, D), _F32),
            jax.ShapeDtypeStruct((1, D), _F32),
            jax.ShapeDtypeStruct((1, D), _F32),
            jax.ShapeDtypeStruct((1, 2), _F32),
            jax.ShapeDtypeStruct((1, 16), _F32),
        ],
        scratch_shapes=[
            pltpu.VMEM((H, N, DH), _F32),
            pltpu.VMEM((H, DH, DH), _F32),
        ],
    )(x2, W1w, b1r, W2w, b2r, g1r, be1r, Wr, brr)

    def _router_sc(probs_hbm, vals_hbm, idx_hbm, pr_v, vals_v, idx_v):
        @pl.when((lax.axis_index("c") == 0) & (lax.axis_index("s") == 0))
        def _():
            pltpu.sync_copy(probs_hbm, pr_v)
            p = pr_v[...]
            ii = lax.iota(jnp.int32, 16)
            sp, si = plsc.sort_key_val(p, ii, descending=True)
            vals_v[...] = sp
            idx_v[...] = si
            pltpu.sync_copy(vals_v, vals_hbm)
            pltpu.sync_copy(idx_v.at[pl.ds(0, 8)], idx_hbm)

    vals16, idx8 = pl.kernel(
        _router_sc,
        out_type=[jax.ShapeDtypeStruct((16,), _F32),
                  jax.ShapeDtypeStruct((8,), jnp.int32)],
        mesh=plsc.VectorSubcoreMesh(core_axis_name="c", subcore_axis_name="s",
                                    num_cores=1),
        scratch_types=[pltpu.VMEM((16,), _F32),
                       pltpu.VMEM((16,), _F32),
                       pltpu.VMEM((16,), jnp.int32)],
        compiler_params=pltpu.CompilerParams(needs_layout_passes=False),
    )(lg16.reshape(16))

    vals2 = vals16.reshape(1, 16)
    bfc3 = bfc.reshape(E, 1, F)
    bproj3 = bproj.reshape(E, 1, D)

    FT = 768
    NF = F // FT
    GE = K * NF

    def _tail(idx_ref, y_ref, cs_ref, css_ref, stats_ref, vals_ref,
              g1_ref, be1_ref, g2_ref, be2_ref,
              wfc_ref, bfc_ref, wproj_ref, bproj_ref, o_ref, m_s):
        g = pl.program_id(0)

        @pl.when(g == 0)
        def _():
            m_s[...] = jnp.zeros_like(m_s)

        @pl.when(g < GE)
        def _():
            j = g // NF
            f = g % NF
            mu = stats_ref[:, 0:1]
            rstd = stats_ref[:, 1:2]
            val = jnp.where(j == 0, vals_ref[:, 0:1], vals_ref[:, 1:2])
            yj = jnp.where(j == 0, y_ref[0:1, :], y_ref[1:2, :])
            x1j = (yj - mu) * rstd * g1_ref[...] + be1_ref[...]
            h = _gelu(_dot_t(x1j, wfc_ref[0]) + bfc_ref[0])
            o = _dot_t(h, wproj_ref[0])
            m_s[...] += val * o

            @pl.when(f == 0)
            def _():
                m_s[...] += val * bproj_ref[0]

        @pl.when(g >= GE)
        def _():
            total = jnp.sum(cs_ref[...])
            mu1 = total / ND
            ssq = jnp.sum(css_ref[...])
            var1 = (ssq - ND * mu1 * mu1) / (ND - 1.0)
            rstd1 = 1.0 / jnp.sqrt(var1 + _EPS)
            a = rstd1 * g1_ref[...]
            c = be1_ref[...] + m_s[...] - mu1 * a
            sz = jnp.sum(a * cs_ref[...] + N * c)
            szz = jnp.sum(a * a * css_ref[...] + 2.0 * a * c * cs_ref[...]
                          + N * c * c)
            mu2 = sz / ND
            var2 = (szz - ND * mu2 * mu2) / (ND - 1.0)
            rstd2 = 1.0 / jnp.sqrt(var2 + _EPS)
            z = y_ref[...] * a + c
            o_ref[...] = (z - mu2) * rstd2 * g2_ref[...] + be2_ref[...]

    def _ytile(g, idx):
        return (jnp.where(g < GE, 0, g - GE), 0)

    def _jj(g):
        return jnp.where(g < GE, g // NF, K - 1)

    def _ff(g):
        return jnp.where(g < GE, g % NF, NF - 1)

    out = pl.pallas_call(
        _tail,
        grid_spec=pltpu.PrefetchScalarGridSpec(
            num_scalar_prefetch=1,
            grid=(GE + NT,),
            in_specs=[
                pl.BlockSpec((TILE, D), _ytile),
                pl.BlockSpec((1, D), lambda g, idx: (0, 0)),
                pl.BlockSpec((1, D), lambda g, idx: (0, 0)),
                pl.BlockSpec((1, 2), lambda g, idx: (0, 0)),
                pl.BlockSpec((1, 16), lambda g, idx: (0, 0)),
                pl.BlockSpec((1, D), lambda g, idx: (0, 0)),
                pl.BlockSpec((1, D), lambda g, idx: (0, 0)),
                pl.BlockSpec((1, D), lambda g, idx: (0, 0)),
                pl.BlockSpec((1, D), lambda g, idx: (0, 0)),
                pl.BlockSpec((1, FT, D),
                             lambda g, idx: (idx[_jj(g)], _ff(g), 0)),
                pl.BlockSpec((1, 1, FT),
                             lambda g, idx: (idx[_jj(g)], 0, _ff(g))),
                pl.BlockSpec((1, D, FT),
                             lambda g, idx: (idx[_jj(g)], 0, _ff(g))),
                pl.BlockSpec((1, 1, D),
                             lambda g, idx: (idx[_jj(g)], 0, 0)),
            ],
            out_specs=pl.BlockSpec((TILE, D), _ytile),
            scratch_shapes=[pltpu.VMEM((1, D), _F32)],
        ),
        out_shape=jax.ShapeDtypeStruct((N, D), _F32),
    )(idx8, y, cs, css, stats2, vals2, g1r, be1r, g2r, be2r,
      Wfc, bfc3, Wproj, bproj3)

    return out.reshape(Bb, N, D)


# speedup vs baseline: 1.0176x; 1.0002x over previous
import math

import jax
import jax.numpy as jnp
from jax import lax
from jax.experimental import pallas as pl
from jax.experimental.pallas import tpu as pltpu
from jax.experimental.pallas import tpu_sc as plsc

_EPS = 1e-12
_F32 = jnp.float32
_BF16 = jnp.bfloat16


def _dot_t(a, b):
    return lax.dot_general(a, b, (((1,), (1,)), ((), ())),
                           preferred_element_type=_F32)


def _gelu(v):
    inner = math.sqrt(2.0 / math.pi) * (v + 0.044715 * v * v * v)
    return 0.5 * v * (1.0 + jnp.tanh(inner))


def kernel(x, W1w, W1b, W2w, W2b, Wfc, bfc, Wproj, bproj, Wr, br, g1, be1, g2, be2):
    Bb, N, D = x.shape
    E, F, _ = Wfc.shape
    H = 12
    DH = D // H
    K = 2
    TILE = 512
    NT = N // TILE
    ND = float(N * D)
    scale = 1.0 / math.sqrt(float(N))

    x2 = x.reshape(N, D)
    b1r = W1b.reshape(1, 3 * D)
    b2r = W2b.reshape(1, D)
    g1r, be1r = g1.reshape(1, D), be1.reshape(1, D)
    g2r, be2r = g2.reshape(1, D), be2.reshape(1, D)
    brr = br.reshape(1, E)

    def _front(x_ref, w1_ref, b1_ref, w2_ref, b2_ref,
               g1_ref, be1_ref, wr_ref, br_ref,
               y_ref, cs_ref, css_ref, stats_ref, lg_ref, v_s, s_s):
        g = pl.program_id(0)

        @pl.when(g == 0)
        def _():
            s_s[...] = jnp.zeros_like(s_s)

        @pl.when(g < NT)
        def _():
            xp = _dot_t(x_ref[...], w1_ref[...]) + b1_ref[...]
            base = pl.multiple_of(g * TILE, TILE)
            for h in range(H):
                v_s[h, pl.ds(base, TILE), :] = \
                    xp[:, 2 * D + h * DH:2 * D + (h + 1) * DH]
                qh = xp[:, h * DH:(h + 1) * DH]
                kh = xp[:, D + h * DH:D + (h + 1) * DH]
                s_s[h] += lax.dot_general(qh, kh, (((0,), (0,)), ((), ())),
                                          preferred_element_type=_F32)

        @pl.when(g >= NT)
        def _():
            i = g - NT
            parts = []
            for t in range(3):
                hh = 3 * i + t
                w = jax.nn.softmax(s_s[hh] * scale, axis=-1)
                parts.append(
                    lax.dot_general(w, v_s[hh], (((1,), (1,)), ((), ())),
                                    preferred_element_type=_F32))
            r = jnp.concatenate(parts, axis=0)
            att = jnp.reshape(r, (TILE, D))
            y = _dot_t(att, w2_ref[...]) + b2_ref[...] + x_ref[...]
            y_ref[...] = y

            @pl.when(i == 0)
            def _():
                cs_ref[...] = jnp.zeros_like(cs_ref)
                css_ref[...] = jnp.zeros_like(css_ref)

            cs_ref[...] += jnp.sum(y, axis=0, keepdims=True)
            css_ref[...] += jnp.sum(y * y, axis=0, keepdims=True)

            @pl.when(i == NT - 1)
            def _():
                total = jnp.sum(cs_ref[...])
                mu = total / ND
                ssq = jnp.sum(css_ref[...])
                var = (ssq - ND * mu * mu) / (ND - 1.0)
                rstd = 1.0 / jnp.sqrt(var + _EPS)
                colmean_x1 = ((cs_ref[...] / N) - mu) * rstd * g1_ref[...] \
                    + be1_ref[...]
                logits = _dot_t(colmean_x1, wr_ref[...]) + br_ref[...]
                probs = jax.nn.softmax(logits, axis=-1)
                pad = jnp.full((1, 16 - E), -1.0, _F32)
                lg_ref[...] = jnp.concatenate([probs, pad], axis=1)
                iota2 = lax.broadcasted_iota(jnp.int32, (1, 2), 1)
                stats_ref[...] = jnp.where(iota2 == 0, mu, rstd)

    def _xtile(g):
        return (jnp.where(g < NT, g, g - NT), 0)

    y, cs, css, stats2, lg16 = pl.pallas_call(
        _front,
        grid=(2 * NT,),
        in_specs=[
            pl.BlockSpec((TILE, D), _xtile),
            pl.BlockSpec((3 * D, D), lambda g: (0, 0)),
            pl.BlockSpec((1, 3 * D), lambda g: (0, 0)),
            pl.BlockSpec((D, D), lambda g: (0, 0)),
            pl.BlockSpec((1, D), lambda g: (0, 0)),
            pl.BlockSpec((1, D), lambda g: (0, 0)),
            pl.BlockSpec((1, D), lambda g: (0, 0)),
            pl.BlockSpec((E, D), lambda g: (0, 0)),
            pl.BlockSpec((1, E), lambda g: (0, 0)),
        ],
        out_specs=[
            pl.BlockSpec((TILE, D), _xtile),
            pl.BlockSpec((1, D), lambda g: (0, 0)),
            pl.BlockSpec((1, D), lambda g: (0, 0)),
            pl.BlockSpec((1, 2), lambda g: (0, 0)),
            pl.BlockSpec((1, 16), lambda g: (0, 0)),
        ],
        out_shape=[
            jax.ShapeDtypeStruct((N, D), _F32),
            jax.ShapeDtypeStruct((1, D), _F32),
            jax.ShapeDtypeStruct((1, D), _F32),
            jax.ShapeDtypeStruct((1, 2), _F32),
            jax.ShapeDtypeStruct((1, 16), _F32),
        ],
        scratch_shapes=[
            pltpu.VMEM((H, N, DH), _F32),
            pltpu.VMEM((H, DH, DH), _F32),
        ],
    )(x2, W1w, b1r, W2w, b2r, g1r, be1r, Wr, brr)

    def _router_sc(probs_hbm, vals_hbm, idx_hbm, pr_v, vals_v, idx_v):
        @pl.when((lax.axis_index("c") == 0) & (lax.axis_index("s") == 0))
        def _():
            pltpu.sync_copy(probs_hbm, pr_v)
            p = pr_v[...]
            ii = lax.iota(jnp.int32, 16)
            sp, si = plsc.sort_key_val(p, ii, descending=True)
            vals_v[...] = sp
            idx_v[...] = si
            pltpu.sync_copy(vals_v, vals_hbm)
            pltpu.sync_copy(idx_v.at[pl.ds(0, 8)], idx_hbm)

    vals16, idx8 = pl.kernel(
        _router_sc,
        out_type=[jax.ShapeDtypeStruct((16,), _F32),
                  jax.ShapeDtypeStruct((8,), jnp.int32)],
        mesh=plsc.VectorSubcoreMesh(core_axis_name="c", subcore_axis_name="s",
                                    num_cores=1, num_subcores=1),
        scratch_types=[pltpu.VMEM((16,), _F32),
                       pltpu.VMEM((16,), _F32),
                       pltpu.VMEM((16,), jnp.int32)],
        compiler_params=pltpu.CompilerParams(needs_layout_passes=False),
    )(lg16.reshape(16))

    vals2 = vals16.reshape(1, 16)
    bfc3 = bfc.reshape(E, 1, F)
    bproj3 = bproj.reshape(E, 1, D)

    FT = 768
    NF = F // FT
    GE = K * NF

    def _tail(idx_ref, y_ref, cs_ref, css_ref, stats_ref, vals_ref,
              g1_ref, be1_ref, g2_ref, be2_ref,
              wfc_ref, bfc_ref, wproj_ref, bproj_ref, o_ref, m_s):
        g = pl.program_id(0)

        @pl.when(g == 0)
        def _():
            m_s[...] = jnp.zeros_like(m_s)

        @pl.when(g < GE)
        def _():
            j = g // NF
            f = g % NF
            mu = stats_ref[:, 0:1]
            rstd = stats_ref[:, 1:2]
            val = jnp.where(j == 0, vals_ref[:, 0:1], vals_ref[:, 1:2])
            yj = jnp.where(j == 0, y_ref[0:1, :], y_ref[1:2, :])
            x1j = (yj - mu) * rstd * g1_ref[...] + be1_ref[...]
            h = _gelu(_dot_t(x1j, wfc_ref[0]) + bfc_ref[0])
            o = _dot_t(h, wproj_ref[0])
            m_s[...] += val * o

            @pl.when(f == 0)
            def _():
                m_s[...] += val * bproj_ref[0]

        @pl.when(g >= GE)
        def _():
            total = jnp.sum(cs_ref[...])
            mu1 = total / ND
            ssq = jnp.sum(css_ref[...])
            var1 = (ssq - ND * mu1 * mu1) / (ND - 1.0)
            rstd1 = 1.0 / jnp.sqrt(var1 + _EPS)
            a = rstd1 * g1_ref[...]
            c = be1_ref[...] + m_s[...] - mu1 * a
            sz = jnp.sum(a * cs_ref[...] + N * c)
            szz = jnp.sum(a * a * css_ref[...] + 2.0 * a * c * cs_ref[...]
                          + N * c * c)
            mu2 = sz / ND
            var2 = (szz - ND * mu2 * mu2) / (ND - 1.0)
            rstd2 = 1.0 / jnp.sqrt(var2 + _EPS)
            z = y_ref[...] * a + c
            o_ref[...] = (z - mu2) * rstd2 * g2_ref[...] + be2_ref[...]

    def _ytile(g, idx):
        return (jnp.where(g < GE, 0, g - GE), 0)

    def _jj(g):
        return jnp.where(g < GE, g // NF, K - 1)

    def _ff(g):
        return jnp.where(g < GE, g % NF, NF - 1)

    out = pl.pallas_call(
        _tail,
        grid_spec=pltpu.PrefetchScalarGridSpec(
            num_scalar_prefetch=1,
            grid=(GE + NT,),
            in_specs=[
                pl.BlockSpec((TILE, D), _ytile),
                pl.BlockSpec((1, D), lambda g, idx: (0, 0)),
                pl.BlockSpec((1, D), lambda g, idx: (0, 0)),
                pl.BlockSpec((1, 2), lambda g, idx: (0, 0)),
                pl.BlockSpec((1, 16), lambda g, idx: (0, 0)),
                pl.BlockSpec((1, D), lambda g, idx: (0, 0)),
                pl.BlockSpec((1, D), lambda g, idx: (0, 0)),
                pl.BlockSpec((1, D), lambda g, idx: (0, 0)),
                pl.BlockSpec((1, D), lambda g, idx: (0, 0)),
                pl.BlockSpec((1, FT, D),
                             lambda g, idx: (idx[_jj(g)], _ff(g), 0)),
                pl.BlockSpec((1, 1, FT),
                             lambda g, idx: (idx[_jj(g)], 0, _ff(g))),
                pl.BlockSpec((1, D, FT),
                             lambda g, idx: (idx[_jj(g)], 0, _ff(g))),
                pl.BlockSpec((1, 1, D),
                             lambda g, idx: (idx[_jj(g)], 0, 0)),
            ],
            out_specs=pl.BlockSpec((TILE, D), _ytile),
            scratch_shapes=[pltpu.VMEM((1, D), _F32)],
        ),
        out_shape=jax.ShapeDtypeStruct((N, D), _F32),
    )(idx8, y, cs, css, stats2, vals2, g1r, be1r, g2r, be2r,
      Wfc, bfc3, Wproj, bproj3)

    return out.reshape(Bb, N, D)
